# baseline (device time: 34042 ns/iter reference)
import jax
import jax.numpy as jnp
from jax import lax
from jax.experimental import pallas as pl
from jax.experimental.pallas import tpu as pltpu

N_DEV = 4
N_TOK = 1024
D_MODEL = 256
N_EXP = 16
H = 512
E_LOCAL = N_EXP // N_DEV
CHUNK = N_TOK // N_DEV


def kernel(x, router_W, route_idx, expert_W):
    def body(x_ref, rw_ref, idx_ref, ew_ref, out_ref,
             part_ref, send_ref, recv_ref, send_sems, recv_sems):
        my = lax.axis_index("i")
        left = lax.rem(my + N_DEV - 1, N_DEV)
        right = lax.rem(my + 1, N_DEV)

        barrier_sem = pltpu.get_barrier_semaphore()
        for nbr in (left, right):
            pl.semaphore_signal(
                barrier_sem, inc=1,
                device_id=(nbr,), device_id_type=pl.DeviceIdType.MESH,
            )
        pl.semaphore_wait(barrier_sem, 2)

        xv = x_ref[:, :]
        scores = jnp.dot(xv, rw_ref[:, :], preferred_element_type=jnp.float32)
        smax = jnp.max(scores, axis=1, keepdims=True)
        pexp = jnp.exp(scores - smax)
        probs = pexp / jnp.sum(pexp, axis=1, keepdims=True)
        idx0 = idx_ref[:, 0:1]
        idx1 = idx_ref[:, 1:2]
        iota = lax.broadcasted_iota(jnp.int32, (N_TOK, N_EXP), 1)
        g0 = jnp.sum(jnp.where(iota == idx0, probs, 0.0), axis=1, keepdims=True)
        g1 = jnp.sum(jnp.where(iota == idx1, probs, 0.0), axis=1, keepdims=True)
        gsum = g0 + g1
        w0 = g0 / gsum
        w1 = g1 / gsum

        xb = xv.astype(jnp.bfloat16)
        part = jnp.zeros((N_TOK, H), jnp.float32)
        for j in range(E_LOCAL):
            e = my * E_LOCAL + j
            wj = (jnp.where(idx0 == e, w0, 0.0)
                  + jnp.where(idx1 == e, w1, 0.0))
            yj = jnp.dot(xb, ew_ref[j].astype(jnp.bfloat16),
                         preferred_element_type=jnp.float32)
            part = part + wj * yj
        part_ref[:, :] = part

        c0 = lax.rem(my + N_DEV - 1, N_DEV)
        acc = part_ref[pl.ds(c0 * CHUNK, CHUNK), :]
        for s in range(N_DEV - 1):
            send_ref[s, :, :] = acc
            rdma = pltpu.make_async_remote_copy(
                src_ref=send_ref.at[s],
                dst_ref=recv_ref.at[s],
                send_sem=send_sems.at[s],
                recv_sem=recv_sems.at[s],
                device_id=(right,),
                device_id_type=pl.DeviceIdType.MESH,
            )
            rdma.start()
            rdma.wait()
            c = lax.rem(my + 2 * N_DEV - 2 - s, N_DEV)
            acc = recv_ref[s, :, :] + part_ref[pl.ds(c * CHUNK, CHUNK), :]
        out_ref[:, :] = acc

    return pl.pallas_call(
        body,
        out_shape=jax.ShapeDtypeStruct((CHUNK, H), jnp.float32),
        in_specs=[pl.BlockSpec(memory_space=pltpu.VMEM)] * 4,
        out_specs=pl.BlockSpec(memory_space=pltpu.VMEM),
        scratch_shapes=[
            pltpu.VMEM((N_TOK, H), jnp.float32),
            pltpu.VMEM((N_DEV - 1, CHUNK, H), jnp.float32),
            pltpu.VMEM((N_DEV - 1, CHUNK, H), jnp.float32),
            pltpu.SemaphoreType.DMA((N_DEV - 1,)),
            pltpu.SemaphoreType.DMA((N_DEV - 1,)),
        ],
        compiler_params=pltpu.CompilerParams(collective_id=0),
    )(x, router_W, route_idx, expert_W)


# device time: 18141 ns/iter; 1.8765x vs baseline; 1.8765x over previous
import jax
import jax.numpy as jnp
from jax import lax
from jax.experimental import pallas as pl
from jax.experimental.pallas import tpu as pltpu

N_DEV = 4
N_TOK = 1024
D_MODEL = 256
N_EXP = 16
H = 512
E_LOCAL = N_EXP // N_DEV
CHUNK = N_TOK // N_DEV


def kernel(x, router_W, route_idx, expert_W):
    def body(x_ref, rw_ref, idx_ref, ew_ref, out_ref,
             xb_ref, ewb_ref, w_ref, send_ref, recv_ref,
             send_sems, recv_sems):
        my = lax.axis_index("i")
        left = lax.rem(my + N_DEV - 1, N_DEV)
        right = lax.rem(my + 1, N_DEV)
        diag = lax.rem(my + 2, N_DEV)

        barrier_sem = pltpu.get_barrier_semaphore()
        for nbr in (left, right):
            pl.semaphore_signal(
                barrier_sem, inc=1,
                device_id=(nbr,), device_id_type=pl.DeviceIdType.MESH,
            )
        pl.semaphore_wait(barrier_sem, 2)

        xv = x_ref[:, :]
        scores = jnp.dot(xv, rw_ref[:, :], preferred_element_type=jnp.float32)
        smax = jnp.max(scores, axis=1, keepdims=True)
        pexp = jnp.exp(scores - smax)
        probs = pexp / jnp.sum(pexp, axis=1, keepdims=True)
        idx0 = idx_ref[:, 0:1]
        idx1 = idx_ref[:, 1:2]
        iota = lax.broadcasted_iota(jnp.int32, (N_TOK, N_EXP), 1)
        g0 = jnp.sum(jnp.where(iota == idx0, probs, 0.0), axis=1, keepdims=True)
        g1 = jnp.sum(jnp.where(iota == idx1, probs, 0.0), axis=1, keepdims=True)
        gsum = g0 + g1
        w_ref[:, 0:1] = g0 / gsum
        w_ref[:, 1:2] = g1 / gsum

        xb_ref[:, :] = xv.astype(jnp.bfloat16)
        for j in range(E_LOCAL):
            ewb_ref[j, :, :] = ew_ref[j].astype(jnp.bfloat16)

        def compute_chunk(c):
            rows = pl.ds(c * CHUNK, CHUNK)
            xc = xb_ref[rows, :]
            i0 = idx_ref[rows, 0:1]
            i1 = idx_ref[rows, 1:2]
            w0 = w_ref[rows, 0:1]
            w1 = w_ref[rows, 1:2]
            acc = jnp.zeros((CHUNK, H), jnp.float32)
            for j in range(E_LOCAL):
                e = my * E_LOCAL + j
                wj = (jnp.where(i0 == e, w0, 0.0)
                      + jnp.where(i1 == e, w1, 0.0))
                yj = jnp.dot(xc, ewb_ref[j],
                             preferred_element_type=jnp.float32)
                acc = acc + wj * yj
            return acc

        rdmas = []
        for slot, dst in enumerate((diag, right, left)):
            send_ref[slot, :, :] = compute_chunk(dst).astype(jnp.bfloat16)
            rdma = pltpu.make_async_remote_copy(
                src_ref=send_ref.at[slot],
                dst_ref=recv_ref.at[my],
                send_sem=send_sems.at[slot],
                recv_sem=recv_sems.at[my],
                device_id=(dst,),
                device_id_type=pl.DeviceIdType.MESH,
            )
            rdma.start()
            rdmas.append(rdma)

        own = compute_chunk(my)

        for src in (left, right, diag):
            recv_wait = pltpu.make_async_remote_copy(
                src_ref=send_ref.at[0],
                dst_ref=recv_ref.at[src],
                send_sem=send_sems.at[0],
                recv_sem=recv_sems.at[src],
                device_id=(src,),
                device_id_type=pl.DeviceIdType.MESH,
            )
            recv_wait.wait_recv()

        total = own
        for src in (left, right, diag):
            total = total + recv_ref[pl.ds(src, 1), :, :][0].astype(jnp.float32)
        out_ref[:, :] = total

        for rdma in rdmas:
            rdma.wait_send()

    return pl.pallas_call(
        body,
        out_shape=jax.ShapeDtypeStruct((CHUNK, H), jnp.float32),
        in_specs=[pl.BlockSpec(memory_space=pltpu.VMEM)] * 4,
        out_specs=pl.BlockSpec(memory_space=pltpu.VMEM),
        scratch_shapes=[
            pltpu.VMEM((N_TOK, D_MODEL), jnp.bfloat16),
            pltpu.VMEM((E_LOCAL, D_MODEL, H), jnp.bfloat16),
            pltpu.VMEM((N_TOK, 2), jnp.float32),
            pltpu.VMEM((N_DEV - 1, CHUNK, H), jnp.bfloat16),
            pltpu.VMEM((N_DEV, CHUNK, H), jnp.bfloat16),
            pltpu.SemaphoreType.DMA((N_DEV - 1,)),
            pltpu.SemaphoreType.DMA((N_DEV,)),
        ],
        compiler_params=pltpu.CompilerParams(collective_id=0),
    )(x, router_W, route_idx, expert_W)


# device time: 16684 ns/iter; 2.0404x vs baseline; 1.0873x over previous
import jax
import jax.numpy as jnp
from jax import lax
from jax.experimental import pallas as pl
from jax.experimental.pallas import tpu as pltpu

N_DEV = 4
N_TOK = 1024
D_MODEL = 256
N_EXP = 16
H = 512
E_LOCAL = N_EXP // N_DEV
CHUNK = N_TOK // N_DEV


def kernel(x, router_W, route_idx, expert_W):
    def body(x_ref, rw_ref, idx_ref, ew_ref, out_ref,
             ewb_ref, send_ref, recv_ref, send_sems, recv_sems):
        my = lax.axis_index("i")
        left = lax.rem(my + N_DEV - 1, N_DEV)
        right = lax.rem(my + 1, N_DEV)
        diag = lax.rem(my + 2, N_DEV)

        barrier_sem = pltpu.get_barrier_semaphore()
        for nbr in (left, right):
            pl.semaphore_signal(
                barrier_sem, inc=1,
                device_id=(nbr,), device_id_type=pl.DeviceIdType.MESH,
            )

        rwb = rw_ref[:, :].astype(jnp.bfloat16)

        def compute_chunk(c, do_cast):
            rows = pl.ds(c * CHUNK, CHUNK)
            xc = x_ref[rows, :].astype(jnp.bfloat16)
            scores = jnp.dot(xc, rwb, preferred_element_type=jnp.float32)
            smax = jnp.max(scores, axis=1, keepdims=True)
            pexp = jnp.exp(scores - smax)
            probs = pexp / jnp.sum(pexp, axis=1, keepdims=True)
            i0 = idx_ref[rows, 0:1]
            i1 = idx_ref[rows, 1:2]
            iota = lax.broadcasted_iota(jnp.int32, (CHUNK, N_EXP), 1)
            g0 = jnp.sum(jnp.where(iota == i0, probs, 0.0),
                         axis=1, keepdims=True)
            g1 = jnp.sum(jnp.where(iota == i1, probs, 0.0),
                         axis=1, keepdims=True)
            gsum = g0 + g1
            w0 = g0 / gsum
            w1 = g1 / gsum
            acc = jnp.zeros((CHUNK, H), jnp.float32)
            for j in range(E_LOCAL):
                if do_cast:
                    ewb_ref[j, :, :] = ew_ref[j].astype(jnp.bfloat16)
                e = my * E_LOCAL + j
                wj = (jnp.where(i0 == e, w0, 0.0)
                      + jnp.where(i1 == e, w1, 0.0))
                yj = jnp.dot(xc, ewb_ref[j, :, :],
                             preferred_element_type=jnp.float32)
                acc = acc + wj * yj
            return acc

        rdmas = []
        for slot, dst in enumerate((diag, right, left)):
            send_ref[slot, :, :] = compute_chunk(dst, slot == 0).astype(
                jnp.bfloat16)
            if slot == 0:
                pl.semaphore_wait(barrier_sem, 2)
            rdma = pltpu.make_async_remote_copy(
                src_ref=send_ref.at[slot],
                dst_ref=recv_ref.at[my],
                send_sem=send_sems.at[slot],
                recv_sem=recv_sems.at[my],
                device_id=(dst,),
                device_id_type=pl.DeviceIdType.MESH,
            )
            rdma.start()
            rdmas.append(rdma)

        total = compute_chunk(my, False)
        for src in (left, right, diag):
            recv_wait = pltpu.make_async_remote_copy(
                src_ref=send_ref.at[0],
                dst_ref=recv_ref.at[src],
                send_sem=send_sems.at[0],
                recv_sem=recv_sems.at[src],
                device_id=(src,),
                device_id_type=pl.DeviceIdType.MESH,
            )
            recv_wait.wait_recv()
            total = total + recv_ref[pl.ds(src, 1), :, :][0].astype(
                jnp.float32)
        out_ref[:, :] = total

        for rdma in rdmas:
            rdma.wait_send()

    return pl.pallas_call(
        body,
        out_shape=jax.ShapeDtypeStruct((CHUNK, H), jnp.float32),
        in_specs=[pl.BlockSpec(memory_space=pltpu.VMEM)] * 4,
        out_specs=pl.BlockSpec(memory_space=pltpu.VMEM),
        scratch_shapes=[
            pltpu.VMEM((E_LOCAL, D_MODEL, H), jnp.bfloat16),
            pltpu.VMEM((N_DEV - 1, CHUNK, H), jnp.bfloat16),
            pltpu.VMEM((N_DEV, CHUNK, H), jnp.bfloat16),
            pltpu.SemaphoreType.DMA((N_DEV - 1,)),
            pltpu.SemaphoreType.DMA((N_DEV,)),
        ],
        compiler_params=pltpu.CompilerParams(collective_id=0),
    )(x, router_W, route_idx, expert_W)
